# baseline (speedup 1.0000x reference)
import functools

import jax
import jax.numpy as jnp
from jax import lax
from jax.experimental import pallas as pl

NP_ = 16
NZ_ = 32
HID_ = 256
BLK_ = 256


def _tc_body(dirs_ref, depth_ref, z_ref, cost_ref, idx_ref, poses_ref,
             w1_ref, b1_ref, w2_ref, b2_ref, w3_ref, b3_ref,
             ps_ref, acc_ref):
    B = dirs_ref.shape[0]
    i = pl.program_id(0)

    idx = idx_ref[...]
    onehot = (idx == lax.broadcasted_iota(jnp.int32, (B, NP_), 1)
              ).astype(jnp.float32)
    prow = jnp.dot(onehot, poses_ref[...],
                   preferred_element_type=jnp.float32)

    d = dirs_ref[...]
    dx, dy, dz = d[:, 0:1], d[:, 1:2], d[:, 2:3]
    dwx = prow[:, 0:1] * dx + prow[:, 1:2] * dy + prow[:, 2:3] * dz
    dwy = prow[:, 4:5] * dx + prow[:, 5:6] * dy + prow[:, 6:7] * dz
    dwz = prow[:, 8:9] * dx + prow[:, 9:10] * dy + prow[:, 10:11] * dz
    tx, ty, tz = prow[:, 3:4], prow[:, 7:8], prow[:, 11:12]

    w1 = w1_ref[...]
    a = tx * w1[0:1, :] + ty * w1[1:2, :] + tz * w1[2:3, :] + b1_ref[...]
    c = dwx * w1[0:1, :] + dwy * w1[1:2, :] + dwz * w1[2:3, :]

    z = z_ref[...]
    w2 = w2_ref[...]
    b2 = b2_ref[...]
    w3 = w3_ref[...]
    b3 = b3_ref[0, 0]
    depth = depth_ref[...]
    sumsq = jnp.zeros((B, 1), dtype=jnp.float32)
    for zi in range(NZ_):
        zcol = z[:, zi:zi + 1]
        h1 = jnp.tanh(a + zcol * c)
        h2 = jnp.tanh(jnp.dot(h1, w2,
                              preferred_element_type=jnp.float32) + b2)
        sdf = jnp.sum(h2 * w3, axis=1, keepdims=True) + b3
        r = sdf - (depth - zcol)
        sumsq += r * r
    per = sumsq * (1.0 / NZ_) * cost_ref[...]
    ps_ref[...] = per

    @pl.when(i == 0)
    def _():
        acc_ref[...] = jnp.zeros_like(acc_ref)
    acc_ref[...] += jnp.sum(per * onehot, axis=0, keepdims=True)


def _fused(dirs, depth2, z_vals, cost2, idx2, pf, w1, b1r, w2, b2r, w3r, b3r,
           interpret=False):
    n = dirs.shape[0]
    nb = n // BLK_
    fixed = lambda i: (0, 0)
    tok = lambda i: (i, 0)
    return pl.pallas_call(
        _tc_body,
        grid=(nb,),
        in_specs=[
            pl.BlockSpec((BLK_, 3), tok),
            pl.BlockSpec((BLK_, 1), tok),
            pl.BlockSpec((BLK_, NZ_), tok),
            pl.BlockSpec((BLK_, 1), tok),
            pl.BlockSpec((BLK_, 1), tok),
            pl.BlockSpec((NP_, 12), fixed),
            pl.BlockSpec((3, HID_), fixed),
            pl.BlockSpec((1, HID_), fixed),
            pl.BlockSpec((HID_, HID_), fixed),
            pl.BlockSpec((1, HID_), fixed),
            pl.BlockSpec((1, HID_), fixed),
            pl.BlockSpec((1, 1), fixed),
        ],
        out_specs=[
            pl.BlockSpec((BLK_, 1), tok),
            pl.BlockSpec((1, NP_), fixed),
        ],
        out_shape=[
            jax.ShapeDtypeStruct((n, 1), jnp.float32),
            jax.ShapeDtypeStruct((1, NP_), jnp.float32),
        ],
        interpret=interpret,
    )(dirs, depth2, z_vals, cost2, idx2, pf, w1, b1r, w2, b2r, w3r, b3r)


def kernel(dirs_C_sample, depth_sample, z_vals, cost_mul, poses,
           W1, b1, W2, b2, W3, b3, indices_b, interpret=False):
    n = dirs_C_sample.shape[0]
    idx2 = indices_b.astype(jnp.int32).reshape(n, 1)
    depth2 = depth_sample.reshape(n, 1)
    cost2 = cost_mul.reshape(n, 1)
    pf = poses.reshape(NP_, 12)
    b1r = b1.reshape(1, HID_)
    b2r = b2.reshape(1, HID_)
    w3r = W3.reshape(1, HID_)
    b3r = b3.reshape(1, 1)
    per, acc = _fused(dirs_C_sample, depth2, z_vals, cost2, idx2, pf,
                      W1, b1r, W2, b2r, w3r, b3r, interpret=interpret)
    return acc.reshape(NP_)


# speedup vs baseline: 1.8497x; 1.8497x over previous
import functools

import jax
import jax.numpy as jnp
from jax import lax
from jax.experimental import pallas as pl

NP_ = 16
NZ_ = 32
HID_ = 256
BLK_ = 256


def _tc_body(dirs_ref, depth_ref, z_ref, cost_ref, idx_ref, poses_ref,
             w1_ref, b1_ref, w2_ref, b2_ref, w3_ref, b3_ref,
             ps_ref, acc_ref):
    B = dirs_ref.shape[0]
    i = pl.program_id(0)

    idx = idx_ref[...]
    onehot = (idx == lax.broadcasted_iota(jnp.int32, (B, NP_), 1)
              ).astype(jnp.float32)
    prow = jnp.dot(onehot, poses_ref[...],
                   preferred_element_type=jnp.float32)

    d = dirs_ref[...]
    dx, dy, dz = d[:, 0:1], d[:, 1:2], d[:, 2:3]
    dwx = prow[:, 0:1] * dx + prow[:, 1:2] * dy + prow[:, 2:3] * dz
    dwy = prow[:, 4:5] * dx + prow[:, 5:6] * dy + prow[:, 6:7] * dz
    dwz = prow[:, 8:9] * dx + prow[:, 9:10] * dy + prow[:, 10:11] * dz
    tx, ty, tz = prow[:, 3:4], prow[:, 7:8], prow[:, 11:12]

    w1 = w1_ref[...]
    a = tx * w1[0:1, :] + ty * w1[1:2, :] + tz * w1[2:3, :] + b1_ref[...]
    c = dwx * w1[0:1, :] + dwy * w1[1:2, :] + dwz * w1[2:3, :]

    z = z_ref[...]
    h1 = jnp.tanh(a[:, None, :] + z[:, :, None] * c[:, None, :])
    h1 = h1.reshape(B * NZ_, HID_).astype(jnp.bfloat16)
    h2 = jnp.tanh(jnp.dot(h1, w2_ref[...],
                          preferred_element_type=jnp.float32) + b2_ref[...])
    s3 = (h2 * w3_ref[...]).reshape(B, NZ_, HID_)
    sdf = jnp.sum(s3, axis=-1) + b3_ref[0, 0]

    res = sdf - (depth_ref[...] - z)
    per = jnp.mean(res * res, axis=1, keepdims=True) * cost_ref[...]
    ps_ref[...] = per

    @pl.when(i == 0)
    def _():
        acc_ref[...] = jnp.zeros_like(acc_ref)
    acc_ref[...] += jnp.sum(per * onehot, axis=0, keepdims=True)


def _fused(dirs, depth2, z_vals, cost2, idx2, pf, w1, b1r, w2, b2r, w3r, b3r,
           interpret=False):
    n = dirs.shape[0]
    nb = n // BLK_
    fixed = lambda i: (0, 0)
    tok = lambda i: (i, 0)
    return pl.pallas_call(
        _tc_body,
        grid=(nb,),
        in_specs=[
            pl.BlockSpec((BLK_, 3), tok),
            pl.BlockSpec((BLK_, 1), tok),
            pl.BlockSpec((BLK_, NZ_), tok),
            pl.BlockSpec((BLK_, 1), tok),
            pl.BlockSpec((BLK_, 1), tok),
            pl.BlockSpec((NP_, 12), fixed),
            pl.BlockSpec((3, HID_), fixed),
            pl.BlockSpec((1, HID_), fixed),
            pl.BlockSpec((HID_, HID_), fixed),
            pl.BlockSpec((1, HID_), fixed),
            pl.BlockSpec((1, HID_), fixed),
            pl.BlockSpec((1, 1), fixed),
        ],
        out_specs=[
            pl.BlockSpec((BLK_, 1), tok),
            pl.BlockSpec((1, NP_), fixed),
        ],
        out_shape=[
            jax.ShapeDtypeStruct((n, 1), jnp.float32),
            jax.ShapeDtypeStruct((1, NP_), jnp.float32),
        ],
        interpret=interpret,
    )(dirs, depth2, z_vals, cost2, idx2, pf, w1, b1r, w2, b2r, w3r, b3r)


def kernel(dirs_C_sample, depth_sample, z_vals, cost_mul, poses,
           W1, b1, W2, b2, W3, b3, indices_b, interpret=False):
    n = dirs_C_sample.shape[0]
    idx2 = indices_b.astype(jnp.int32).reshape(n, 1)
    depth2 = depth_sample.reshape(n, 1)
    cost2 = cost_mul.reshape(n, 1)
    pf = poses.reshape(NP_, 12)
    b1r = b1.reshape(1, HID_)
    b2r = b2.reshape(1, HID_)
    w3r = W3.reshape(1, HID_)
    b3r = b3.reshape(1, 1)
    per, acc = _fused(dirs_C_sample, depth2, z_vals, cost2, idx2, pf,
                      W1, b1r, W2.astype(jnp.bfloat16), b2r, w3r, b3r,
                      interpret=interpret)
    return acc.reshape(NP_)


# speedup vs baseline: 1.9223x; 1.0393x over previous
import functools

import jax
import jax.numpy as jnp
from jax import lax
from jax.experimental import pallas as pl

NP_ = 16
NZ_ = 32
HID_ = 256
BLK_ = 256


def _tc_body(dirs_ref, depth_ref, z_ref, cost_ref, idx_ref, poses_ref,
             w1_ref, b1_ref, w2_ref, b2_ref, w3_ref, b3_ref,
             ps_ref, acc_ref):
    B = dirs_ref.shape[0]
    i = pl.program_id(0)

    idx = idx_ref[...]
    onehot = (idx == lax.broadcasted_iota(jnp.int32, (B, NP_), 1)
              ).astype(jnp.float32)
    prow = jnp.dot(onehot, poses_ref[...],
                   preferred_element_type=jnp.float32)

    d = dirs_ref[...]
    dx, dy, dz = d[:, 0:1], d[:, 1:2], d[:, 2:3]
    dwx = prow[:, 0:1] * dx + prow[:, 1:2] * dy + prow[:, 2:3] * dz
    dwy = prow[:, 4:5] * dx + prow[:, 5:6] * dy + prow[:, 6:7] * dz
    dwz = prow[:, 8:9] * dx + prow[:, 9:10] * dy + prow[:, 10:11] * dz
    tx, ty, tz = prow[:, 3:4], prow[:, 7:8], prow[:, 11:12]

    w1 = w1_ref[...]
    a = tx * w1[0:1, :] + ty * w1[1:2, :] + tz * w1[2:3, :] + b1_ref[...]
    c = dwx * w1[0:1, :] + dwy * w1[1:2, :] + dwz * w1[2:3, :]

    z = z_ref[...]
    ab = a.astype(jnp.bfloat16)
    cb = c.astype(jnp.bfloat16)
    zb = z.astype(jnp.bfloat16)
    pre = ab[:, None, :] + zb[:, :, None] * cb[:, None, :]
    h1 = jnp.tanh(pre.reshape(B * NZ_, HID_)).astype(jnp.bfloat16)
    h2 = jnp.tanh(jnp.dot(h1, w2_ref[...],
                          preferred_element_type=jnp.float32) + b2_ref[...])
    s3 = (h2 * w3_ref[...]).reshape(B, NZ_, HID_)
    sdf = jnp.sum(s3, axis=-1) + b3_ref[0, 0]

    res = sdf - (depth_ref[...] - z)
    per = jnp.mean(res * res, axis=1, keepdims=True) * cost_ref[...]
    ps_ref[...] = per

    @pl.when(i == 0)
    def _():
        acc_ref[...] = jnp.zeros_like(acc_ref)
    acc_ref[...] += jnp.sum(per * onehot, axis=0, keepdims=True)


def _fused(dirs, depth2, z_vals, cost2, idx2, pf, w1, b1r, w2, b2r, w3r, b3r,
           interpret=False):
    n = dirs.shape[0]
    nb = n // BLK_
    fixed = lambda i: (0, 0)
    tok = lambda i: (i, 0)
    return pl.pallas_call(
        _tc_body,
        grid=(nb,),
        in_specs=[
            pl.BlockSpec((BLK_, 3), tok),
            pl.BlockSpec((BLK_, 1), tok),
            pl.BlockSpec((BLK_, NZ_), tok),
            pl.BlockSpec((BLK_, 1), tok),
            pl.BlockSpec((BLK_, 1), tok),
            pl.BlockSpec((NP_, 12), fixed),
            pl.BlockSpec((3, HID_), fixed),
            pl.BlockSpec((1, HID_), fixed),
            pl.BlockSpec((HID_, HID_), fixed),
            pl.BlockSpec((1, HID_), fixed),
            pl.BlockSpec((1, HID_), fixed),
            pl.BlockSpec((1, 1), fixed),
        ],
        out_specs=[
            pl.BlockSpec((BLK_, 1), tok),
            pl.BlockSpec((1, NP_), fixed),
        ],
        out_shape=[
            jax.ShapeDtypeStruct((n, 1), jnp.float32),
            jax.ShapeDtypeStruct((1, NP_), jnp.float32),
        ],
        interpret=interpret,
    )(dirs, depth2, z_vals, cost2, idx2, pf, w1, b1r, w2, b2r, w3r, b3r)


def kernel(dirs_C_sample, depth_sample, z_vals, cost_mul, poses,
           W1, b1, W2, b2, W3, b3, indices_b, interpret=False):
    n = dirs_C_sample.shape[0]
    idx2 = indices_b.astype(jnp.int32).reshape(n, 1)
    depth2 = depth_sample.reshape(n, 1)
    cost2 = cost_mul.reshape(n, 1)
    pf = poses.reshape(NP_, 12)
    b1r = b1.reshape(1, HID_)
    b2r = b2.reshape(1, HID_)
    w3r = W3.reshape(1, HID_)
    b3r = b3.reshape(1, 1)
    per, acc = _fused(dirs_C_sample, depth2, z_vals, cost2, idx2, pf,
                      W1, b1r, W2.astype(jnp.bfloat16), b2r, w3r, b3r,
                      interpret=interpret)
    return acc.reshape(NP_)


# speedup vs baseline: 2.0130x; 1.0472x over previous
import functools

import jax
import jax.numpy as jnp
from jax import lax
from jax.experimental import pallas as pl

NP_ = 16
NZ_ = 32
HID_ = 256
BLK_ = 512


def _tc_body(dirs_ref, depth_ref, z_ref, cost_ref, idx_ref, poses_ref,
             w1_ref, b1_ref, w2_ref, b2_ref, w3_ref, b3_ref,
             ps_ref, acc_ref):
    B = dirs_ref.shape[0]
    i = pl.program_id(0)

    idx = idx_ref[...]
    onehot = (idx == lax.broadcasted_iota(jnp.int32, (B, NP_), 1)
              ).astype(jnp.float32)
    prow = jnp.dot(onehot, poses_ref[...],
                   preferred_element_type=jnp.float32)

    d = dirs_ref[...]
    dx, dy, dz = d[:, 0:1], d[:, 1:2], d[:, 2:3]
    dwx = prow[:, 0:1] * dx + prow[:, 1:2] * dy + prow[:, 2:3] * dz
    dwy = prow[:, 4:5] * dx + prow[:, 5:6] * dy + prow[:, 6:7] * dz
    dwz = prow[:, 8:9] * dx + prow[:, 9:10] * dy + prow[:, 10:11] * dz
    tx, ty, tz = prow[:, 3:4], prow[:, 7:8], prow[:, 11:12]

    w1 = w1_ref[...]
    a = tx * w1[0:1, :] + ty * w1[1:2, :] + tz * w1[2:3, :] + b1_ref[...]
    c = dwx * w1[0:1, :] + dwy * w1[1:2, :] + dwz * w1[2:3, :]

    z = z_ref[...]
    ab = a.astype(jnp.bfloat16)
    cb = c.astype(jnp.bfloat16)
    zb = z.astype(jnp.bfloat16)
    pre = ab[:, None, :] + zb[:, :, None] * cb[:, None, :]
    h1 = jnp.tanh(pre.reshape(B * NZ_, HID_)).astype(jnp.bfloat16)
    h2 = jnp.tanh(jnp.dot(h1, w2_ref[...],
                          preferred_element_type=jnp.float32) + b2_ref[...])
    s3 = (h2 * w3_ref[...]).reshape(B, NZ_, HID_)
    sdf = jnp.sum(s3, axis=-1) + b3_ref[0, 0]

    res = sdf - (depth_ref[...] - z)
    per = jnp.mean(res * res, axis=1, keepdims=True) * cost_ref[...]
    ps_ref[...] = per

    @pl.when(i == 0)
    def _():
        acc_ref[...] = jnp.zeros_like(acc_ref)
    acc_ref[...] += jnp.sum(per * onehot, axis=0, keepdims=True)


def _fused(dirs, depth2, z_vals, cost2, idx2, pf, w1, b1r, w2, b2r, w3r, b3r,
           interpret=False):
    n = dirs.shape[0]
    nb = n // BLK_
    fixed = lambda i: (0, 0)
    tok = lambda i: (i, 0)
    return pl.pallas_call(
        _tc_body,
        grid=(nb,),
        in_specs=[
            pl.BlockSpec((BLK_, 3), tok),
            pl.BlockSpec((BLK_, 1), tok),
            pl.BlockSpec((BLK_, NZ_), tok),
            pl.BlockSpec((BLK_, 1), tok),
            pl.BlockSpec((BLK_, 1), tok),
            pl.BlockSpec((NP_, 12), fixed),
            pl.BlockSpec((3, HID_), fixed),
            pl.BlockSpec((1, HID_), fixed),
            pl.BlockSpec((HID_, HID_), fixed),
            pl.BlockSpec((1, HID_), fixed),
            pl.BlockSpec((1, HID_), fixed),
            pl.BlockSpec((1, 1), fixed),
        ],
        out_specs=[
            pl.BlockSpec((BLK_, 1), tok),
            pl.BlockSpec((1, NP_), fixed),
        ],
        out_shape=[
            jax.ShapeDtypeStruct((n, 1), jnp.float32),
            jax.ShapeDtypeStruct((1, NP_), jnp.float32),
        ],
        interpret=interpret,
    )(dirs, depth2, z_vals, cost2, idx2, pf, w1, b1r, w2, b2r, w3r, b3r)


def kernel(dirs_C_sample, depth_sample, z_vals, cost_mul, poses,
           W1, b1, W2, b2, W3, b3, indices_b, interpret=False):
    n = dirs_C_sample.shape[0]
    idx2 = indices_b.astype(jnp.int32).reshape(n, 1)
    depth2 = depth_sample.reshape(n, 1)
    cost2 = cost_mul.reshape(n, 1)
    pf = poses.reshape(NP_, 12)
    b1r = b1.reshape(1, HID_)
    b2r = b2.reshape(1, HID_)
    w3r = W3.reshape(1, HID_)
    b3r = b3.reshape(1, 1)
    per, acc = _fused(dirs_C_sample, depth2, z_vals, cost2, idx2, pf,
                      W1, b1r, W2.astype(jnp.bfloat16), b2r, w3r, b3r,
                      interpret=interpret)
    return acc.reshape(NP_)
